# baseline (device time: 132637 ns/iter reference)
import jax
import jax.numpy as jnp
from jax import lax
from jax.experimental import pallas as pl
from jax.experimental.pallas import tpu as pltpu

N_DEV = 16
SQ = 512
D = 1024
SKV = 2048
DH = 128
HQ_LOCAL = 8
GQA = 4
CHUNK = SQ // N_DEV
SCALE = 0.08838834764831843


def kernel(x, Wq, Wo, K_ext, V_ext):
    my = lax.axis_index("i")

    kv0 = my * (HQ_LOCAL // GQA)
    k_loc = lax.dynamic_slice_in_dim(K_ext[0], kv0, HQ_LOCAL // GQA, axis=1)
    v_loc = lax.dynamic_slice_in_dim(V_ext[0], kv0, HQ_LOCAL // GQA, axis=1)
    k_loc = k_loc.transpose(1, 0, 2).astype(jnp.bfloat16)
    v_loc = v_loc.transpose(1, 0, 2).astype(jnp.bfloat16)

    x2 = x.reshape(SQ, D).astype(jnp.bfloat16)
    wq = Wq.astype(jnp.bfloat16)
    wo = Wo.astype(jnp.bfloat16)

    def body(x_ref, wq_ref, wo_ref, k_ref, v_ref, out_ref,
             acc_ref, comm_ref, rs_send, rs_recv, ag_send, ag_recv):
        my_pos = lax.axis_index("i")
        left = lax.rem(my_pos - 1 + N_DEV, N_DEV)
        right = lax.rem(my_pos + 1, N_DEV)

        q_all = jnp.dot(x_ref[:, :], wq_ref[:, :],
                        preferred_element_type=jnp.float32)
        heads = []
        for h in range(HQ_LOCAL):
            q_h = (q_all[:, h * DH:(h + 1) * DH] * SCALE).astype(jnp.bfloat16)
            kv = h // GQA
            s = lax.dot_general(
                q_h, k_ref[kv],
                (((1,), (1,)), ((), ())),
                preferred_element_type=jnp.float32,
            )
            m = jnp.max(s, axis=1, keepdims=True)
            e = jnp.exp(s - m)
            l = jnp.sum(e, axis=1, keepdims=True)
            o = lax.dot_general(
                e.astype(jnp.bfloat16), v_ref[kv],
                (((1,), (0,)), ((), ())),
                preferred_element_type=jnp.float32,
            )
            heads.append((o / l).astype(jnp.bfloat16))
        attn = jnp.concatenate(heads, axis=1)
        acc_ref[:, :] = jnp.dot(attn, wo_ref[:, :],
                                preferred_element_type=jnp.float32)

        barrier_sem = pltpu.get_barrier_semaphore()
        for nbr in (left, right):
            pl.semaphore_signal(
                barrier_sem, inc=1,
                device_id=(nbr,), device_id_type=pl.DeviceIdType.MESH,
            )
        pl.semaphore_wait(barrier_sem, 2)

        for h in range(N_DEV - 1):
            send_idx = lax.rem(my_pos - h + N_DEV, N_DEV)
            rdma = pltpu.make_async_remote_copy(
                src_ref=acc_ref.at[pl.ds(send_idx * CHUNK, CHUNK), :],
                dst_ref=comm_ref.at[h],
                send_sem=rs_send.at[h],
                recv_sem=rs_recv.at[h],
                device_id=(right,),
                device_id_type=pl.DeviceIdType.MESH,
            )
            rdma.start()
            rdma.wait()
            recv_idx = lax.rem(my_pos - 1 - h + 2 * N_DEV, N_DEV)
            sl = pl.ds(recv_idx * CHUNK, CHUNK)
            acc_ref[sl, :] = acc_ref[sl, :] + comm_ref[h, :, :]

        own = lax.rem(my_pos + 1, N_DEV)
        out_ref[pl.ds(own * CHUNK, CHUNK), :] = acc_ref[
            pl.ds(own * CHUNK, CHUNK), :]

        for g in range(N_DEV - 1):
            send_idx = lax.rem(my_pos + 1 - g + 2 * N_DEV, N_DEV)
            rdma = pltpu.make_async_remote_copy(
                src_ref=out_ref.at[pl.ds(send_idx * CHUNK, CHUNK), :],
                dst_ref=out_ref.at[pl.ds(send_idx * CHUNK, CHUNK), :],
                send_sem=ag_send.at[g],
                recv_sem=ag_recv.at[g],
                device_id=(right,),
                device_id_type=pl.DeviceIdType.MESH,
            )
            rdma.start()
            rdma.wait()

    out = pl.pallas_call(
        body,
        out_shape=jax.ShapeDtypeStruct((SQ, D), jnp.float32),
        in_specs=[
            pl.BlockSpec(memory_space=pltpu.VMEM),
            pl.BlockSpec(memory_space=pltpu.VMEM),
            pl.BlockSpec(memory_space=pltpu.VMEM),
            pl.BlockSpec(memory_space=pltpu.VMEM),
            pl.BlockSpec(memory_space=pltpu.VMEM),
        ],
        out_specs=pl.BlockSpec(memory_space=pltpu.VMEM),
        scratch_shapes=[
            pltpu.VMEM((SQ, D), jnp.float32),
            pltpu.VMEM((N_DEV - 1, CHUNK, D), jnp.float32),
            pltpu.SemaphoreType.DMA((N_DEV - 1,)),
            pltpu.SemaphoreType.DMA((N_DEV - 1,)),
            pltpu.SemaphoreType.DMA((N_DEV - 1,)),
            pltpu.SemaphoreType.DMA((N_DEV - 1,)),
        ],
        compiler_params=pltpu.CompilerParams(collective_id=0),
    )(x2, wq, wo, k_loc, v_loc)

    return out.reshape(1, SQ, D)


# device time: 57780 ns/iter; 2.2956x vs baseline; 2.2956x over previous
import jax
import jax.numpy as jnp
from jax import lax
from jax.experimental import pallas as pl
from jax.experimental.pallas import tpu as pltpu

N_DEV = 16
SQ = 512
D = 1024
SKV = 2048
DH = 128
HQ_LOCAL = 8
GQA = 4
CHUNK = SQ // N_DEV
SCALE = 0.08838834764831843


def kernel(x, Wq, Wo, K_ext, V_ext):
    my = lax.axis_index("i")

    kv0 = my * (HQ_LOCAL // GQA)
    k_loc = lax.dynamic_slice_in_dim(K_ext[0], kv0, HQ_LOCAL // GQA, axis=1)
    v_loc = lax.dynamic_slice_in_dim(V_ext[0], kv0, HQ_LOCAL // GQA, axis=1)
    k_loc = k_loc.transpose(1, 0, 2).astype(jnp.bfloat16)
    v_loc = v_loc.transpose(1, 0, 2).astype(jnp.bfloat16)

    x2 = x.reshape(SQ, D).astype(jnp.bfloat16)
    wq = Wq.astype(jnp.bfloat16)
    wo = Wo.astype(jnp.bfloat16)

    def body(x_ref, wq_ref, wo_ref, k_ref, v_ref, out_ref,
             acc_ref, comm_ref, a_send, a_recv, b_send, b_recv):
        me = lax.axis_index("i")

        barrier_sem = pltpu.get_barrier_semaphore()
        for o in range(1, N_DEV):
            peer = lax.rem(me + o, N_DEV)
            pl.semaphore_signal(
                barrier_sem, inc=1,
                device_id=(peer,), device_id_type=pl.DeviceIdType.MESH,
            )
        pl.semaphore_wait(barrier_sem, N_DEV - 1)

        q_all = jnp.dot(x_ref[:, :], wq_ref[:, :],
                        preferred_element_type=jnp.float32)
        heads = []
        for h in range(HQ_LOCAL):
            q_h = (q_all[:, h * DH:(h + 1) * DH] * SCALE).astype(jnp.bfloat16)
            kv = h // GQA
            s = lax.dot_general(
                q_h, k_ref[kv],
                (((1,), (1,)), ((), ())),
                preferred_element_type=jnp.float32,
            )
            m = jnp.max(s, axis=1, keepdims=True)
            e = jnp.exp(s - m)
            l = jnp.sum(e, axis=1, keepdims=True)
            o_h = lax.dot_general(
                e.astype(jnp.bfloat16), v_ref[kv],
                (((1,), (0,)), ((), ())),
                preferred_element_type=jnp.float32,
            )
            heads.append((o_h / l).astype(jnp.bfloat16))
        attn = jnp.concatenate(heads, axis=1)
        acc_ref[:, :] = jnp.dot(attn, wo_ref[:, :],
                                preferred_element_type=jnp.float32
                                ).astype(jnp.bfloat16)

        a_rdmas = []
        for o in range(1, N_DEV):
            peer = lax.rem(me + o, N_DEV)
            rdma = pltpu.make_async_remote_copy(
                src_ref=acc_ref.at[pl.ds(peer * CHUNK, CHUNK), :],
                dst_ref=comm_ref.at[me],
                send_sem=a_send.at[peer],
                recv_sem=a_recv.at[me],
                device_id=(peer,),
                device_id_type=pl.DeviceIdType.MESH,
            )
            rdma.start()
            a_rdmas.append(rdma)

        comm_ref[me, :, :] = acc_ref[pl.ds(me * CHUNK, CHUNK), :]
        for o in range(1, N_DEV):
            s = lax.rem(me + N_DEV - o, N_DEV)
            recv = pltpu.make_async_remote_copy(
                src_ref=comm_ref.at[s],
                dst_ref=comm_ref.at[s],
                send_sem=a_send.at[s],
                recv_sem=a_recv.at[s],
                device_id=(s,),
                device_id_type=pl.DeviceIdType.MESH,
            )
            recv.wait_recv()

        reduced = jnp.sum(comm_ref[:, :, :].astype(jnp.float32), axis=0)
        out_ref[pl.ds(me * CHUNK, CHUNK), :] = reduced.astype(jnp.bfloat16)

        b_rdmas = []
        for o in range(1, N_DEV):
            peer = lax.rem(me + o, N_DEV)
            rdma = pltpu.make_async_remote_copy(
                src_ref=out_ref.at[pl.ds(me * CHUNK, CHUNK), :],
                dst_ref=out_ref.at[pl.ds(me * CHUNK, CHUNK), :],
                send_sem=b_send.at[peer],
                recv_sem=b_recv.at[me],
                device_id=(peer,),
                device_id_type=pl.DeviceIdType.MESH,
            )
            rdma.start()
            b_rdmas.append(rdma)

        for o in range(1, N_DEV):
            s = lax.rem(me + N_DEV - o, N_DEV)
            recv = pltpu.make_async_remote_copy(
                src_ref=out_ref.at[pl.ds(s * CHUNK, CHUNK), :],
                dst_ref=out_ref.at[pl.ds(s * CHUNK, CHUNK), :],
                send_sem=b_send.at[s],
                recv_sem=b_recv.at[s],
                device_id=(s,),
                device_id_type=pl.DeviceIdType.MESH,
            )
            recv.wait_recv()

        for rdma in a_rdmas + b_rdmas:
            rdma.wait_send()

    out = pl.pallas_call(
        body,
        out_shape=jax.ShapeDtypeStruct((SQ, D), jnp.bfloat16),
        in_specs=[
            pl.BlockSpec(memory_space=pltpu.VMEM),
            pl.BlockSpec(memory_space=pltpu.VMEM),
            pl.BlockSpec(memory_space=pltpu.VMEM),
            pl.BlockSpec(memory_space=pltpu.VMEM),
            pl.BlockSpec(memory_space=pltpu.VMEM),
        ],
        out_specs=pl.BlockSpec(memory_space=pltpu.VMEM),
        scratch_shapes=[
            pltpu.VMEM((SQ, D), jnp.bfloat16),
            pltpu.VMEM((N_DEV, CHUNK, D), jnp.bfloat16),
            pltpu.SemaphoreType.DMA((N_DEV,)),
            pltpu.SemaphoreType.DMA((N_DEV,)),
            pltpu.SemaphoreType.DMA((N_DEV,)),
            pltpu.SemaphoreType.DMA((N_DEV,)),
        ],
        compiler_params=pltpu.CompilerParams(collective_id=0),
    )(x2, wq, wo, k_loc, v_loc)

    return out.reshape(1, SQ, D)
